# baseline (speedup 1.0000x reference)
import functools

import jax
import jax.numpy as jnp
from jax import lax
from jax.experimental import pallas as pl
from jax.experimental.pallas import tpu as pltpu
from jax.experimental.pallas import tpu_sc as plsc

_SLOPE = 10.0
_BUDGET = 48
_LANES = 16


def _softplus_body(s_ref, p_ref):
    p_ref[...] = jax.nn.softplus(_SLOPE * s_ref[...]) / _SLOPE


def _softplus_row(sampler):
    return pl.pallas_call(
        _softplus_body,
        out_shape=jax.ShapeDtypeStruct(sampler.shape, jnp.float32),
    )(sampler)


def _lane_reduce(red_v, v, op):
    ii = lax.iota(jnp.int32, _LANES)
    for sh in (8, 4, 2, 1):
        red_v[...] = v
        v = op(v, plsc.load_gather(red_v, [ii ^ sh]))
    return v


def _policy_body(nbatch, width, prob_hbm, mask_hbm, thresh_hbm,
                 row_hbm, fp_hbm,
                 prob_v, m_v, t_v, row_v, fp_v, red_v):
    nchunk = width // _LANES
    wid = lax.axis_index("s") * 2 + lax.axis_index("c")

    @pl.when(wid < nbatch)
    def _():
        b = wid
        pltpu.sync_copy(prob_hbm, prob_v)
        pltpu.sync_copy(mask_hbm.at[b], m_v)
        pltpu.sync_copy(thresh_hbm.at[b], t_v)

        dmax = (1.0 - m_v[pl.ds(0, _LANES)]) * prob_v[pl.ds(0, _LANES)]
        for i in range(1, nchunk):
            sl = pl.ds(_LANES * i, _LANES)
            dmax = jnp.maximum(dmax, (1.0 - m_v[sl]) * prob_v[sl])
        denom = _lane_reduce(red_v, dmax, jnp.maximum)

        cnt = jnp.zeros((_LANES,), jnp.float32)
        s1 = jnp.zeros((_LANES,), jnp.float32)
        for i in range(nchunk):
            sl = pl.ds(_LANES * i, _LANES)
            pm = prob_v[sl] / denom
            m = m_v[sl]
            mp = pm * (1.0 - m)
            z = (m == 0.0).astype(jnp.float32)
            cnt = cnt + z
            s1 = s1 + mp * z
        count = _lane_reduce(red_v, cnt, jnp.add)
        xbar = _lane_reduce(red_v, s1, jnp.add) / count
        sparsity = _BUDGET / count
        r = sparsity / xbar
        beta = (1.0 - sparsity) / (1.0 - xbar)
        le = (r <= 1.0).astype(jnp.float32)

        for i in range(nchunk):
            sl = pl.ds(_LANES * i, _LANES)
            pm = prob_v[sl] / denom
            m = m_v[sl]
            mp = pm * (1.0 - m)
            resc = le * mp * r + (1.0 - le) * (1.0 - (1.0 - mp) * beta)
            mp2 = jnp.where(m == 0.0, resc, mp)
            binv = (mp2 > t_v[sl]).astype(jnp.float32)
            row_v[sl] = m + binv
            fp_v[sl] = mp2

        pltpu.sync_copy(row_v, row_hbm.at[b])
        pltpu.sync_copy(fp_v, fp_hbm.at[b])


def _policy_rows(prob, mask1d, thresh):
    nbatch, width = mask1d.shape
    mesh = plsc.VectorSubcoreMesh(
        core_axis_name="c", subcore_axis_name="s", num_cores=2,
        num_subcores=16)
    return pl.kernel(
        functools.partial(_policy_body, nbatch, width),
        out_type=(
            jax.ShapeDtypeStruct((nbatch, width), jnp.float32),
            jax.ShapeDtypeStruct((nbatch, width), jnp.float32),
        ),
        mesh=mesh,
        compiler_params=pltpu.CompilerParams(needs_layout_passes=False),
        scratch_types=[
            pltpu.VMEM((width,), jnp.float32),
            pltpu.VMEM((width,), jnp.float32),
            pltpu.VMEM((width,), jnp.float32),
            pltpu.VMEM((width,), jnp.float32),
            pltpu.VMEM((width,), jnp.float32),
            pltpu.VMEM((_LANES,), jnp.float32),
        ],
    )(prob, mask1d, thresh)


def _apply_body(rows_ref, fp_ref, k_ref, mk_ref, out_ref, mo_ref, fpb_ref,
                mcp_ref):
    _, nc, h, _, width = k_ref.shape
    b = pl.program_id(0)
    r = rows_ref[pl.ds(b, 1), :]
    out_ref[...] = k_ref[...] * r[:, None, None, None, :]
    mo_ref[0, :, 0, :] = jnp.broadcast_to(r, (h, width))
    fpb_ref[0, :, 0, :] = jnp.broadcast_to(fp_ref[pl.ds(b, 1), :], (h, width))
    mcp_ref[...] = mk_ref[...]


def _apply_mask(ktr, mask_t, row, fprow):
    nbatch, ncoil, h, _, width = ktr.shape
    grid = (nbatch,)
    return pl.pallas_call(
        _apply_body,
        grid=grid,
        in_specs=[
            pl.BlockSpec((nbatch, width), lambda b: (0, 0)),
            pl.BlockSpec((nbatch, width), lambda b: (0, 0)),
            pl.BlockSpec((1, ncoil, h, 2, width),
                         lambda b: (b, 0, 0, 0, 0)),
            pl.BlockSpec((1, 1, h, 1, width), lambda b: (b, 0, 0, 0, 0)),
        ],
        out_specs=[
            pl.BlockSpec((1, ncoil, h, 2, width),
                         lambda b: (b, 0, 0, 0, 0)),
            pl.BlockSpec((1, h, 1, width), lambda b: (b, 0, 0, 0)),
            pl.BlockSpec((1, h, 1, width), lambda b: (b, 0, 0, 0)),
            pl.BlockSpec((1, 1, h, 1, width), lambda b: (b, 0, 0, 0, 0)),
        ],
        out_shape=[
            jax.ShapeDtypeStruct(
                (nbatch, ncoil, h, 2, width), jnp.float32),
            jax.ShapeDtypeStruct((nbatch, h, 1, width), jnp.float32),
            jax.ShapeDtypeStruct((nbatch, h, 1, width), jnp.float32),
            jax.ShapeDtypeStruct((nbatch, 1, h, 1, width), jnp.float32),
        ],
        compiler_params=pltpu.CompilerParams(
            vmem_limit_bytes=56 * 1024 * 1024),
    )(row, fprow, ktr, mask_t)


def kernel(mask, kspace, sampler, thresh):
    nbatch, ncoil, h, width = kspace.shape[:4]
    mask1d = mask[:, 0, 0, :, 0]
    prob = _softplus_row(sampler).reshape(width)
    row, fprow = _policy_rows(prob, mask1d, thresh)
    ktr = jnp.swapaxes(kspace, 3, 4)
    mask_t = jnp.swapaxes(mask, 3, 4)
    masked_t, mo, fpb, mcp_t = _apply_mask(ktr, mask_t, row, fprow)
    masked_kspace = jnp.swapaxes(masked_t, 3, 4)
    mask_copy = jnp.swapaxes(mcp_t, 3, 4)
    mask_out = mo.reshape(nbatch, 1, h, width, 1)
    final_prob = fpb.reshape(nbatch, 1, h, width, 1)
    return masked_kspace, mask_copy, mask_out, final_prob


# speedup vs baseline: 1.0178x; 1.0178x over previous
import functools

import jax
import jax.numpy as jnp
from jax import lax
from jax.experimental import pallas as pl
from jax.experimental.pallas import tpu as pltpu
from jax.experimental.pallas import tpu_sc as plsc

_SLOPE = 10.0
_BUDGET = 48
_LANES = 16


def _softplus_body(s_ref, p_ref):
    p_ref[...] = jax.nn.softplus(_SLOPE * s_ref[...]) / _SLOPE


def _softplus_row(sampler):
    return pl.pallas_call(
        _softplus_body,
        out_shape=jax.ShapeDtypeStruct(sampler.shape, jnp.float32),
    )(sampler)


def _lane_reduce(red_v, v, op):
    ii = lax.iota(jnp.int32, _LANES)
    for sh in (8, 4, 2, 1):
        red_v[...] = v
        v = op(v, plsc.load_gather(red_v, [ii ^ sh]))
    return v


def _policy_body(nbatch, width, prob_hbm, mask_hbm, thresh_hbm,
                 row_hbm, fp_hbm,
                 prob_v, m_v, t_v, row_v, fp_v, red_v):
    nchunk = width // _LANES
    wid = lax.axis_index("s") * 2 + lax.axis_index("c")

    @pl.when(wid < nbatch)
    def _():
        b = wid
        pltpu.sync_copy(prob_hbm, prob_v)
        pltpu.sync_copy(mask_hbm.at[b], m_v)
        pltpu.sync_copy(thresh_hbm.at[b], t_v)

        dmax = (1.0 - m_v[pl.ds(0, _LANES)]) * prob_v[pl.ds(0, _LANES)]
        for i in range(1, nchunk):
            sl = pl.ds(_LANES * i, _LANES)
            dmax = jnp.maximum(dmax, (1.0 - m_v[sl]) * prob_v[sl])
        denom = _lane_reduce(red_v, dmax, jnp.maximum)

        cnt = jnp.zeros((_LANES,), jnp.float32)
        s1 = jnp.zeros((_LANES,), jnp.float32)
        for i in range(nchunk):
            sl = pl.ds(_LANES * i, _LANES)
            pm = prob_v[sl] / denom
            m = m_v[sl]
            mp = pm * (1.0 - m)
            z = (m == 0.0).astype(jnp.float32)
            cnt = cnt + z
            s1 = s1 + mp * z
        count = _lane_reduce(red_v, cnt, jnp.add)
        xbar = _lane_reduce(red_v, s1, jnp.add) / count
        sparsity = _BUDGET / count
        r = sparsity / xbar
        beta = (1.0 - sparsity) / (1.0 - xbar)
        le = (r <= 1.0).astype(jnp.float32)

        for i in range(nchunk):
            sl = pl.ds(_LANES * i, _LANES)
            pm = prob_v[sl] / denom
            m = m_v[sl]
            mp = pm * (1.0 - m)
            resc = le * mp * r + (1.0 - le) * (1.0 - (1.0 - mp) * beta)
            mp2 = jnp.where(m == 0.0, resc, mp)
            binv = (mp2 > t_v[sl]).astype(jnp.float32)
            row_v[sl] = m + binv
            fp_v[sl] = mp2

        pltpu.sync_copy(row_v, row_hbm.at[b])
        pltpu.sync_copy(fp_v, fp_hbm.at[b])


def _policy_rows(prob, mask1d, thresh):
    nbatch, width = mask1d.shape
    mesh = plsc.VectorSubcoreMesh(
        core_axis_name="c", subcore_axis_name="s", num_cores=2,
        num_subcores=16)
    return pl.kernel(
        functools.partial(_policy_body, nbatch, width),
        out_type=(
            jax.ShapeDtypeStruct((nbatch, width), jnp.float32),
            jax.ShapeDtypeStruct((nbatch, width), jnp.float32),
        ),
        mesh=mesh,
        compiler_params=pltpu.CompilerParams(needs_layout_passes=False),
        scratch_types=[
            pltpu.VMEM((width,), jnp.float32),
            pltpu.VMEM((width,), jnp.float32),
            pltpu.VMEM((width,), jnp.float32),
            pltpu.VMEM((width,), jnp.float32),
            pltpu.VMEM((width,), jnp.float32),
            pltpu.VMEM((_LANES,), jnp.float32),
        ],
    )(prob, mask1d, thresh)


def _apply_body(rows_ref, fp_ref, k_ref, out_ref, mo_ref, fpb_ref):
    _, nc, h, _, width = k_ref.shape
    b = pl.program_id(0)
    r = rows_ref[pl.ds(b, 1), :]
    out_ref[...] = k_ref[...] * r[:, None, None, None, :]
    mo_ref[0, :, 0, :] = jnp.broadcast_to(r, (h, width))
    fpb_ref[0, :, 0, :] = jnp.broadcast_to(fp_ref[pl.ds(b, 1), :], (h, width))


def _apply_mask(ktr, row, fprow):
    nbatch, ncoil, h, _, width = ktr.shape
    grid = (nbatch,)
    return pl.pallas_call(
        _apply_body,
        grid=grid,
        in_specs=[
            pl.BlockSpec((nbatch, width), lambda b: (0, 0)),
            pl.BlockSpec((nbatch, width), lambda b: (0, 0)),
            pl.BlockSpec((1, ncoil, h, 2, width),
                         lambda b: (b, 0, 0, 0, 0)),
        ],
        out_specs=[
            pl.BlockSpec((1, ncoil, h, 2, width),
                         lambda b: (b, 0, 0, 0, 0)),
            pl.BlockSpec((1, h, 1, width), lambda b: (b, 0, 0, 0)),
            pl.BlockSpec((1, h, 1, width), lambda b: (b, 0, 0, 0)),
        ],
        out_shape=[
            jax.ShapeDtypeStruct(
                (nbatch, ncoil, h, 2, width), jnp.float32),
            jax.ShapeDtypeStruct((nbatch, h, 1, width), jnp.float32),
            jax.ShapeDtypeStruct((nbatch, h, 1, width), jnp.float32),
        ],
        compiler_params=pltpu.CompilerParams(
            vmem_limit_bytes=56 * 1024 * 1024),
    )(row, fprow, ktr)


def kernel(mask, kspace, sampler, thresh):
    nbatch, ncoil, h, width = kspace.shape[:4]
    mask1d = mask[:, 0, 0, :, 0]
    prob = _softplus_row(sampler).reshape(width)
    row, fprow = _policy_rows(prob, mask1d, thresh)
    ktr = jnp.swapaxes(kspace, 3, 4)
    masked_t, mo, fpb = _apply_mask(ktr, row, fprow)
    masked_kspace = jnp.swapaxes(masked_t, 3, 4)
    mask_out = mo.reshape(nbatch, 1, h, width, 1)
    final_prob = fpb.reshape(nbatch, 1, h, width, 1)
    return masked_kspace, mask, mask_out, final_prob


# speedup vs baseline: 1.0287x; 1.0107x over previous
import functools

import jax
import jax.numpy as jnp
from jax import lax
from jax.experimental import pallas as pl
from jax.experimental.pallas import tpu as pltpu
from jax.experimental.pallas import tpu_sc as plsc

_SLOPE = 10.0
_BUDGET = 48
_LANES = 16


def _softplus_body(s_ref, p_ref):
    p_ref[...] = jax.nn.softplus(_SLOPE * s_ref[...]) / _SLOPE


def _softplus_row(sampler):
    return pl.pallas_call(
        _softplus_body,
        out_shape=jax.ShapeDtypeStruct(sampler.shape, jnp.float32),
    )(sampler)


def _lane_reduce(red_v, v, op):
    ii = lax.iota(jnp.int32, _LANES)
    for sh in (8, 4, 2, 1):
        red_v[...] = v
        v = op(v, plsc.load_gather(red_v, [ii ^ sh]))
    return v


def _policy_body(nbatch, width, prob_hbm, mask_hbm, thresh_hbm,
                 row_hbm, fp_hbm,
                 prob_v, m_v, t_v, row_v, fp_v, red_v, sem1, sem2, sem3):
    nchunk = width // _LANES
    wid = lax.axis_index("s") * 2 + lax.axis_index("c")

    @pl.when(wid < nbatch)
    def _():
        b = wid
        cp1 = pltpu.async_copy(prob_hbm, prob_v, sem1)
        cp2 = pltpu.async_copy(mask_hbm.at[b], m_v, sem2)
        cp3 = pltpu.async_copy(thresh_hbm.at[b], t_v, sem3)
        cp1.wait()
        cp2.wait()
        cp3.wait()

        dmax = (1.0 - m_v[pl.ds(0, _LANES)]) * prob_v[pl.ds(0, _LANES)]
        for i in range(1, nchunk):
            sl = pl.ds(_LANES * i, _LANES)
            dmax = jnp.maximum(dmax, (1.0 - m_v[sl]) * prob_v[sl])
        denom = _lane_reduce(red_v, dmax, jnp.maximum)

        cnt = jnp.zeros((_LANES,), jnp.float32)
        s1 = jnp.zeros((_LANES,), jnp.float32)
        for i in range(nchunk):
            sl = pl.ds(_LANES * i, _LANES)
            pm = prob_v[sl] / denom
            m = m_v[sl]
            mp = pm * (1.0 - m)
            z = (m == 0.0).astype(jnp.float32)
            cnt = cnt + z
            s1 = s1 + mp * z
        count = _lane_reduce(red_v, cnt, jnp.add)
        xbar = _lane_reduce(red_v, s1, jnp.add) / count
        sparsity = _BUDGET / count
        r = sparsity / xbar
        beta = (1.0 - sparsity) / (1.0 - xbar)
        le = (r <= 1.0).astype(jnp.float32)

        for i in range(nchunk):
            sl = pl.ds(_LANES * i, _LANES)
            pm = prob_v[sl] / denom
            m = m_v[sl]
            mp = pm * (1.0 - m)
            resc = le * mp * r + (1.0 - le) * (1.0 - (1.0 - mp) * beta)
            mp2 = jnp.where(m == 0.0, resc, mp)
            binv = (mp2 > t_v[sl]).astype(jnp.float32)
            row_v[sl] = m + binv
            fp_v[sl] = mp2

        co1 = pltpu.async_copy(row_v, row_hbm.at[b], sem1)
        co2 = pltpu.async_copy(fp_v, fp_hbm.at[b], sem2)
        co1.wait()
        co2.wait()


def _policy_rows(prob, mask1d, thresh):
    nbatch, width = mask1d.shape
    mesh = plsc.VectorSubcoreMesh(
        core_axis_name="c", subcore_axis_name="s", num_cores=2,
        num_subcores=16)
    return pl.kernel(
        functools.partial(_policy_body, nbatch, width),
        out_type=(
            jax.ShapeDtypeStruct((nbatch, width), jnp.float32),
            jax.ShapeDtypeStruct((nbatch, width), jnp.float32),
        ),
        mesh=mesh,
        compiler_params=pltpu.CompilerParams(needs_layout_passes=False),
        scratch_types=[
            pltpu.VMEM((width,), jnp.float32),
            pltpu.VMEM((width,), jnp.float32),
            pltpu.VMEM((width,), jnp.float32),
            pltpu.VMEM((width,), jnp.float32),
            pltpu.VMEM((width,), jnp.float32),
            pltpu.VMEM((_LANES,), jnp.float32),
            pltpu.SemaphoreType.DMA,
            pltpu.SemaphoreType.DMA,
            pltpu.SemaphoreType.DMA,
        ],
    )(prob, mask1d, thresh)


def _apply_body(rows_ref, fp_ref, k_ref, out_ref, mo_ref, fpb_ref):
    _, nc, h, _, width = k_ref.shape
    b = pl.program_id(0)
    r = rows_ref[pl.ds(b, 1), :]
    out_ref[...] = k_ref[...] * r[:, None, None, None, :]
    mo_ref[0, :, 0, :] = jnp.broadcast_to(r, (h, width))
    fpb_ref[0, :, 0, :] = jnp.broadcast_to(fp_ref[pl.ds(b, 1), :], (h, width))


def _apply_mask(ktr, row, fprow):
    nbatch, ncoil, h, _, width = ktr.shape
    grid = (nbatch,)
    return pl.pallas_call(
        _apply_body,
        grid=grid,
        in_specs=[
            pl.BlockSpec((nbatch, width), lambda b: (0, 0)),
            pl.BlockSpec((nbatch, width), lambda b: (0, 0)),
            pl.BlockSpec((1, ncoil, h, 2, width),
                         lambda b: (b, 0, 0, 0, 0)),
        ],
        out_specs=[
            pl.BlockSpec((1, ncoil, h, 2, width),
                         lambda b: (b, 0, 0, 0, 0)),
            pl.BlockSpec((1, h, 1, width), lambda b: (b, 0, 0, 0)),
            pl.BlockSpec((1, h, 1, width), lambda b: (b, 0, 0, 0)),
        ],
        out_shape=[
            jax.ShapeDtypeStruct(
                (nbatch, ncoil, h, 2, width), jnp.float32),
            jax.ShapeDtypeStruct((nbatch, h, 1, width), jnp.float32),
            jax.ShapeDtypeStruct((nbatch, h, 1, width), jnp.float32),
        ],
        compiler_params=pltpu.CompilerParams(
            vmem_limit_bytes=56 * 1024 * 1024),
    )(row, fprow, ktr)


def kernel(mask, kspace, sampler, thresh):
    nbatch, ncoil, h, width = kspace.shape[:4]
    mask1d = mask[:, 0, 0, :, 0]
    prob = _softplus_row(sampler).reshape(width)
    row, fprow = _policy_rows(prob, mask1d, thresh)
    ktr = jnp.swapaxes(kspace, 3, 4)
    masked_t, mo, fpb = _apply_mask(ktr, row, fprow)
    masked_kspace = jnp.swapaxes(masked_t, 3, 4)
    mask_out = mo.reshape(nbatch, 1, h, width, 1)
    final_prob = fpb.reshape(nbatch, 1, h, width, 1)
    return masked_kspace, mask, mask_out, final_prob


# speedup vs baseline: 1.0408x; 1.0118x over previous
import functools

import jax
import jax.numpy as jnp
from jax import lax
from jax.experimental import pallas as pl
from jax.experimental.pallas import tpu as pltpu
from jax.experimental.pallas import tpu_sc as plsc

_SLOPE = 10.0
_BUDGET = 48
_LANES = 16


def _softplus_body(s_ref, mk_ref, p_ref, m_ref):
    p_ref[...] = (jax.nn.softplus(_SLOPE * s_ref[...]) / _SLOPE)[0]
    m_ref[...] = mk_ref[:, 0, 0, 0, :]


def _prep_rows(sampler, mask_t):
    nbatch = mask_t.shape[0]
    width = sampler.shape[1]
    return pl.pallas_call(
        _softplus_body,
        grid=(1,),
        in_specs=[
            pl.BlockSpec((1, width), lambda i: (0, 0)),
            pl.BlockSpec((nbatch, 1, 1, 1, width),
                         lambda i: (0, 0, 0, 0, 0)),
        ],
        out_specs=[
            pl.BlockSpec((width,), lambda i: (0,)),
            pl.BlockSpec((nbatch, width), lambda i: (0, 0)),
        ],
        out_shape=[
            jax.ShapeDtypeStruct((width,), jnp.float32),
            jax.ShapeDtypeStruct((nbatch, width), jnp.float32),
        ],
    )(sampler, mask_t)


def _lane_reduce(red_v, v, op):
    ii = lax.iota(jnp.int32, _LANES)
    for sh in (8, 4, 2, 1):
        red_v[...] = v
        v = op(v, plsc.load_gather(red_v, [ii ^ sh]))
    return v


def _policy_body(nbatch, width, prob_hbm, mask_hbm, thresh_hbm,
                 row_hbm, fp_hbm,
                 prob_v, m_v, t_v, row_v, fp_v, red_v, sem1, sem2, sem3):
    nchunk = width // _LANES
    wid = lax.axis_index("s") * 2 + lax.axis_index("c")

    @pl.when(wid < nbatch)
    def _():
        b = wid
        cp1 = pltpu.async_copy(prob_hbm, prob_v, sem1)
        cp2 = pltpu.async_copy(mask_hbm.at[b], m_v, sem2)
        cp3 = pltpu.async_copy(thresh_hbm.at[b], t_v, sem3)
        cp1.wait()
        cp2.wait()
        cp3.wait()

        dmax = (1.0 - m_v[pl.ds(0, _LANES)]) * prob_v[pl.ds(0, _LANES)]
        for i in range(1, nchunk):
            sl = pl.ds(_LANES * i, _LANES)
            dmax = jnp.maximum(dmax, (1.0 - m_v[sl]) * prob_v[sl])
        denom = _lane_reduce(red_v, dmax, jnp.maximum)

        cnt = jnp.zeros((_LANES,), jnp.float32)
        s1 = jnp.zeros((_LANES,), jnp.float32)
        for i in range(nchunk):
            sl = pl.ds(_LANES * i, _LANES)
            pm = prob_v[sl] / denom
            m = m_v[sl]
            mp = pm * (1.0 - m)
            z = (m == 0.0).astype(jnp.float32)
            cnt = cnt + z
            s1 = s1 + mp * z
        count = _lane_reduce(red_v, cnt, jnp.add)
        xbar = _lane_reduce(red_v, s1, jnp.add) / count
        sparsity = _BUDGET / count
        r = sparsity / xbar
        beta = (1.0 - sparsity) / (1.0 - xbar)
        le = (r <= 1.0).astype(jnp.float32)

        for i in range(nchunk):
            sl = pl.ds(_LANES * i, _LANES)
            pm = prob_v[sl] / denom
            m = m_v[sl]
            mp = pm * (1.0 - m)
            resc = le * mp * r + (1.0 - le) * (1.0 - (1.0 - mp) * beta)
            mp2 = jnp.where(m == 0.0, resc, mp)
            binv = (mp2 > t_v[sl]).astype(jnp.float32)
            row_v[sl] = m + binv
            fp_v[sl] = mp2

        co1 = pltpu.async_copy(row_v, row_hbm.at[b], sem1)
        co2 = pltpu.async_copy(fp_v, fp_hbm.at[b], sem2)
        co1.wait()
        co2.wait()


def _policy_rows(prob, mask1d, thresh):
    nbatch, width = mask1d.shape
    mesh = plsc.VectorSubcoreMesh(
        core_axis_name="c", subcore_axis_name="s", num_cores=2,
        num_subcores=16)
    return pl.kernel(
        functools.partial(_policy_body, nbatch, width),
        out_type=(
            jax.ShapeDtypeStruct((nbatch, width), jnp.float32),
            jax.ShapeDtypeStruct((nbatch, width), jnp.float32),
        ),
        mesh=mesh,
        compiler_params=pltpu.CompilerParams(needs_layout_passes=False),
        scratch_types=[
            pltpu.VMEM((width,), jnp.float32),
            pltpu.VMEM((width,), jnp.float32),
            pltpu.VMEM((width,), jnp.float32),
            pltpu.VMEM((width,), jnp.float32),
            pltpu.VMEM((width,), jnp.float32),
            pltpu.VMEM((_LANES,), jnp.float32),
            pltpu.SemaphoreType.DMA,
            pltpu.SemaphoreType.DMA,
            pltpu.SemaphoreType.DMA,
        ],
    )(prob, mask1d, thresh)


def _apply_body(rows_ref, fp_ref, k_ref, out_ref, mo_ref, fpb_ref):
    _, nc, h, _, width = k_ref.shape
    b = pl.program_id(0)
    r = rows_ref[pl.ds(b, 1), :]
    out_ref[...] = k_ref[...] * r[:, None, None, None, :]
    mo_ref[0, :, 0, :] = jnp.broadcast_to(r, (h, width))
    fpb_ref[0, :, 0, :] = jnp.broadcast_to(fp_ref[pl.ds(b, 1), :], (h, width))


def _apply_mask(ktr, row, fprow):
    nbatch, ncoil, h, _, width = ktr.shape
    grid = (nbatch,)
    return pl.pallas_call(
        _apply_body,
        grid=grid,
        in_specs=[
            pl.BlockSpec((nbatch, width), lambda b: (0, 0)),
            pl.BlockSpec((nbatch, width), lambda b: (0, 0)),
            pl.BlockSpec((1, ncoil, h, 2, width),
                         lambda b: (b, 0, 0, 0, 0)),
        ],
        out_specs=[
            pl.BlockSpec((1, ncoil, h, 2, width),
                         lambda b: (b, 0, 0, 0, 0)),
            pl.BlockSpec((1, h, 1, width), lambda b: (b, 0, 0, 0)),
            pl.BlockSpec((1, h, 1, width), lambda b: (b, 0, 0, 0)),
        ],
        out_shape=[
            jax.ShapeDtypeStruct(
                (nbatch, ncoil, h, 2, width), jnp.float32),
            jax.ShapeDtypeStruct((nbatch, h, 1, width), jnp.float32),
            jax.ShapeDtypeStruct((nbatch, h, 1, width), jnp.float32),
        ],
        compiler_params=pltpu.CompilerParams(
            vmem_limit_bytes=56 * 1024 * 1024),
    )(row, fprow, ktr)


def kernel(mask, kspace, sampler, thresh):
    nbatch, ncoil, h, width = kspace.shape[:4]
    mask_t = jnp.swapaxes(mask, 3, 4)
    prob, mask1d = _prep_rows(sampler, mask_t)
    row, fprow = _policy_rows(prob, mask1d, thresh)
    ktr = jnp.swapaxes(kspace, 3, 4)
    masked_t, mo, fpb = _apply_mask(ktr, row, fprow)
    masked_kspace = jnp.swapaxes(masked_t, 3, 4)
    mask_out = mo.reshape(nbatch, 1, h, width, 1)
    final_prob = fpb.reshape(nbatch, 1, h, width, 1)
    return masked_kspace, mask, mask_out, final_prob
